# baseline (speedup 1.0000x reference)
import functools

import jax
import jax.numpy as jnp
from jax import lax
from jax.experimental import pallas as pl
from jax.experimental.pallas import tpu as pltpu
from jax.experimental.pallas import tpu_sc as plsc

N = 10000
D = 128
E = 320000

NC = 2
NS = 16
NW = NC * NS
CHUNK = 128
NCHUNK = 80
EPW = NCHUNK * CHUNK
E_PAD = NW * EPW
NP = 10240
ROWS_PT = NP // NS
PG = 16
NPAGE = NCHUNK // PG

_sc_mesh = plsc.VectorSubcoreMesh(core_axis_name="c", subcore_axis_name="s")


@functools.partial(
    pl.kernel,
    out_type=jax.ShapeDtypeStruct((NC, NP), jnp.float32),
    mesh=_sc_mesh,
    scratch_types=[
        pltpu.VMEM((NCHUNK, CHUNK), jnp.int32),
        pltpu.VMEM((CHUNK,), jnp.float32),
        pltpu.VMEM((ROWS_PT,), jnp.float32),
        pltpu.VMEM_SHARED((NP,), jnp.float32),
        pltpu.SemaphoreType.DMA,
    ],
)
def _sc_degree(dst_hbm, out_hbm, idx_all, ones_v, zeros_v, deg_sh, sem):
    cid = lax.axis_index("c")
    sid = lax.axis_index("s")
    wid = sid * NC + cid

    pltpu.async_copy(dst_hbm.at[wid], idx_all, sem)
    for j in range(CHUNK // 16):
        ones_v[pl.ds(j * 16, 16)] = jnp.ones((16,), jnp.float32)
    for j in range(ROWS_PT // 16):
        zeros_v[pl.ds(j * 16, 16)] = jnp.zeros((16,), jnp.float32)
    pltpu.sync_copy(zeros_v, deg_sh.at[pl.ds(sid * ROWS_PT, ROWS_PT)])
    pltpu.make_async_copy(dst_hbm.at[wid], idx_all, sem).wait()
    plsc.subcore_barrier()

    def body(k, _):
        pltpu.sync_copy(ones_v, deg_sh.at[idx_all.at[k]], add=True)
        return 0

    lax.fori_loop(0, NCHUNK, body, 0)
    plsc.subcore_barrier()
    pltpu.sync_copy(deg_sh.at[pl.ds(sid * ROWS_PT, ROWS_PT)],
                    out_hbm.at[cid, pl.ds(sid * ROWS_PT, ROWS_PT)])


@functools.partial(
    pl.kernel,
    out_type=jax.ShapeDtypeStruct((NC, NP, D), jnp.float32),
    mesh=_sc_mesh,
    scratch_types=[
        pltpu.VMEM((EPW,), jnp.int32),
        pltpu.VMEM((2, PG, CHUNK), jnp.int32),
        pltpu.VMEM((CHUNK, D), jnp.float32),
        pltpu.VMEM((CHUNK, D), jnp.float32),
        pltpu.VMEM_SHARED((NP, D), jnp.float32),
        pltpu.SemaphoreType.DMA,
        pltpu.SemaphoreType.DMA,
        pltpu.SemaphoreType.DMA,
        pltpu.SemaphoreType.DMA,
    ],
)
def _sc_agg(g_hbm, src_hbm, dst_hbm, out_hbm, sidx_all, didx_pg, rows0,
            rows1, acc_sh, sem0, sem1, psem0, psem1):
    cid = lax.axis_index("c")
    sid = lax.axis_index("s")
    wid = sid * NC + cid

    pltpu.async_copy(src_hbm.at[pl.ds(wid * EPW, EPW)], sidx_all, sem0)
    pltpu.async_copy(dst_hbm.at[wid, pl.ds(0, PG)], didx_pg.at[0], psem0)

    def zero_row(r, _):
        for j in range(D // 16):
            rows0[r, pl.ds(j * 16, 16)] = jnp.zeros((16,), jnp.float32)
        return 0

    lax.fori_loop(0, CHUNK, zero_row, 0)
    for i in range(ROWS_PT // CHUNK):
        pltpu.sync_copy(rows0, acc_sh.at[pl.ds(sid * ROWS_PT + i * CHUNK, CHUNK)])
    pltpu.make_async_copy(src_hbm.at[pl.ds(wid * EPW, EPW)], sidx_all,
                          sem0).wait()
    pltpu.make_async_copy(dst_hbm.at[wid, pl.ds(0, PG)], didx_pg.at[0],
                          psem0).wait()
    plsc.subcore_barrier()

    def sidx(k):
        return sidx_all.at[pl.ds(k * CHUNK, CHUNK)]

    pltpu.async_copy(g_hbm.at[sidx(0)], rows0, sem0)

    for p in range(NPAGE):
        pg = p % 2
        psem_nxt = psem1 if (1 - pg) else psem0
        if p + 1 < NPAGE:
            pltpu.async_copy(dst_hbm.at[wid, pl.ds((p + 1) * PG, PG)],
                             didx_pg.at[1 - pg], psem_nxt)
        if p > 0:
            psem_cur = psem1 if pg else psem0
            pltpu.make_async_copy(dst_hbm.at[wid, pl.ds(p * PG, PG)],
                                  didx_pg.at[pg], psem_cur).wait()

        npairs = PG // 2 if p + 1 < NPAGE else PG // 2 - 1

        def body(t, _, p=p, pg=pg):
            k0 = p * PG + 2 * t
            pltpu.async_copy(g_hbm.at[sidx(k0 + 1)], rows1, sem1)
            pltpu.make_async_copy(g_hbm.at[sidx(k0)], rows0, sem0).wait()
            pltpu.sync_copy(rows0, acc_sh.at[didx_pg.at[pg, 2 * t]], add=True)
            pltpu.async_copy(g_hbm.at[sidx(k0 + 2)], rows0, sem0)
            pltpu.make_async_copy(g_hbm.at[sidx(k0 + 1)], rows1, sem1).wait()
            pltpu.sync_copy(rows1, acc_sh.at[didx_pg.at[pg, 2 * t + 1]],
                            add=True)
            return 0

        lax.fori_loop(0, npairs, body, 0)

    lpg = (NPAGE - 1) % 2
    pltpu.async_copy(g_hbm.at[sidx(NCHUNK - 1)], rows1, sem1)
    pltpu.make_async_copy(g_hbm.at[sidx(NCHUNK - 2)], rows0, sem0).wait()
    pltpu.sync_copy(rows0, acc_sh.at[didx_pg.at[lpg, PG - 2]], add=True)
    pltpu.make_async_copy(g_hbm.at[sidx(NCHUNK - 1)], rows1, sem1).wait()
    pltpu.sync_copy(rows1, acc_sh.at[didx_pg.at[lpg, PG - 1]], add=True)

    plsc.subcore_barrier()
    pltpu.sync_copy(acc_sh.at[pl.ds(sid * ROWS_PT, ROWS_PT)],
                    out_hbm.at[cid, pl.ds(sid * ROWS_PT, ROWS_PT)])


BLK = 1000


def _dinv(degs_ref):
    return lax.rsqrt(degs_ref[...] + 1.0)


def _gemm_scale_body(x_ref, w_ref, degp_ref, g_ref):
    h = jnp.dot(x_ref[...], w_ref[...], preferred_element_type=jnp.float32)
    g_ref[...] = h * _dinv(degp_ref)


def _tc_gemm_scale(x, w, degp):
    return pl.pallas_call(
        _gemm_scale_body,
        grid=(N // BLK,),
        in_specs=[
            pl.BlockSpec((BLK, D), lambda i: (i, 0)),
            pl.BlockSpec((D, D), lambda i: (0, 0)),
            pl.BlockSpec((BLK, 1), lambda i: (i, 0)),
        ],
        out_specs=pl.BlockSpec((BLK, D), lambda i: (i, 0)),
        out_shape=jax.ShapeDtypeStruct((N, D), jnp.float32),
    )(x, w, degp)


def _mid_body(p_ref, g_ref, degp_ref, b_ref, w_ref, o_ref):
    dinv = _dinv(degp_ref)
    acc = p_ref[0] + p_ref[1] + g_ref[...]
    x2 = jnp.maximum(acc * dinv + b_ref[...], 0.0)
    h2 = jnp.dot(x2, w_ref[...], preferred_element_type=jnp.float32)
    o_ref[...] = h2 * dinv


def _tc_mid(parts, g, degp, b, w):
    return pl.pallas_call(
        _mid_body,
        grid=(N // BLK,),
        in_specs=[
            pl.BlockSpec((NC, BLK, D), lambda i: (0, i, 0)),
            pl.BlockSpec((BLK, D), lambda i: (i, 0)),
            pl.BlockSpec((BLK, 1), lambda i: (i, 0)),
            pl.BlockSpec((1, D), lambda i: (0, 0)),
            pl.BlockSpec((D, D), lambda i: (0, 0)),
        ],
        out_specs=pl.BlockSpec((BLK, D), lambda i: (i, 0)),
        out_shape=jax.ShapeDtypeStruct((N, D), jnp.float32),
    )(parts, g, degp, b, w)


def _final_body(p_ref, g_ref, degp_ref, b_ref, o_ref):
    acc = p_ref[0] + p_ref[1] + g_ref[...]
    o_ref[...] = acc * _dinv(degp_ref) + b_ref[...]


def _tc_final(parts, g, degp, b):
    return pl.pallas_call(
        _final_body,
        grid=(N // BLK,),
        in_specs=[
            pl.BlockSpec((NC, BLK, D), lambda i: (0, i, 0)),
            pl.BlockSpec((BLK, D), lambda i: (i, 0)),
            pl.BlockSpec((BLK, 1), lambda i: (i, 0)),
            pl.BlockSpec((1, D), lambda i: (0, 0)),
        ],
        out_specs=pl.BlockSpec((BLK, D), lambda i: (i, 0)),
        out_shape=jax.ShapeDtypeStruct((N, D), jnp.float32),
    )(parts, g, degp, b)


def kernel(edge_index, node_features, W1, b1, W2, b2):
    ppw = EPW - E // NW
    pad_src = jnp.broadcast_to(
        (jnp.arange(ppw, dtype=jnp.int32) * 41) % N, (NW, ppw))
    pad_dst = jnp.broadcast_to(
        N + (jnp.arange(ppw, dtype=jnp.int32) % (NP - N)), (NW, ppw))
    src = jnp.concatenate(
        [edge_index[0].reshape(NW, E // NW), pad_src], axis=1).reshape(-1)
    dst = jnp.concatenate(
        [edge_index[1].reshape(NW, E // NW), pad_dst], axis=1).reshape(
            NW, NCHUNK, CHUNK)

    degp = _sc_degree(dst)
    degs = (degp[0] + degp[1]).reshape(NP, 1)
    g1 = _tc_gemm_scale(node_features, W1, degs)
    p1 = _sc_agg(g1, src, dst)
    g2 = _tc_mid(p1, g1, degs, b1.reshape(1, D), W2)
    p2 = _sc_agg(g2, src, dst)
    return _tc_final(p2, g2, degs, b2.reshape(1, D))


# speedup vs baseline: 1.0239x; 1.0239x over previous
import functools

import jax
import jax.numpy as jnp
from jax import lax
from jax.experimental import pallas as pl
from jax.experimental.pallas import tpu as pltpu
from jax.experimental.pallas import tpu_sc as plsc

N = 10000
D = 128
E = 320000

NC = 2
NS = 16
NW = NC * NS
CHUNK = 128
NCHUNK = 80
EPW = NCHUNK * CHUNK
E_PAD = NW * EPW
NP = 10240
ROWS_PT = NP // NS
PG = 16
NPAGE = NCHUNK // PG

_sc_mesh = plsc.VectorSubcoreMesh(core_axis_name="c", subcore_axis_name="s")


@functools.partial(
    pl.kernel,
    out_type=jax.ShapeDtypeStruct((NC, NP), jnp.float32),
    mesh=_sc_mesh,
    scratch_types=[
        pltpu.VMEM((NCHUNK, CHUNK), jnp.int32),
        pltpu.VMEM((CHUNK,), jnp.float32),
        pltpu.VMEM((ROWS_PT,), jnp.float32),
        pltpu.VMEM_SHARED((NP,), jnp.float32),
        pltpu.SemaphoreType.DMA,
    ],
)
def _sc_degree(dst_hbm, out_hbm, idx_all, ones_v, zeros_v, deg_sh, sem):
    cid = lax.axis_index("c")
    sid = lax.axis_index("s")
    wid = sid * NC + cid

    pltpu.async_copy(dst_hbm.at[wid], idx_all, sem)
    for j in range(CHUNK // 16):
        ones_v[pl.ds(j * 16, 16)] = jnp.ones((16,), jnp.float32)
    for j in range(ROWS_PT // 16):
        zeros_v[pl.ds(j * 16, 16)] = jnp.zeros((16,), jnp.float32)
    pltpu.sync_copy(zeros_v, deg_sh.at[pl.ds(sid * ROWS_PT, ROWS_PT)])
    pltpu.make_async_copy(dst_hbm.at[wid], idx_all, sem).wait()
    plsc.subcore_barrier()

    def body(k, _):
        pltpu.sync_copy(ones_v, deg_sh.at[idx_all.at[k]], add=True)
        return 0

    lax.fori_loop(0, NCHUNK, body, 0)
    plsc.subcore_barrier()
    pltpu.sync_copy(deg_sh.at[pl.ds(sid * ROWS_PT, ROWS_PT)],
                    out_hbm.at[cid, pl.ds(sid * ROWS_PT, ROWS_PT)])


@functools.partial(
    pl.kernel,
    out_type=jax.ShapeDtypeStruct((NC, NP, D), jnp.float32),
    mesh=_sc_mesh,
    scratch_types=[
        pltpu.VMEM((EPW,), jnp.int32),
        pltpu.VMEM((2, PG, CHUNK), jnp.int32),
        pltpu.VMEM((CHUNK, D), jnp.float32),
        pltpu.VMEM((CHUNK, D), jnp.float32),
        pltpu.VMEM_SHARED((NP, D), jnp.float32),
        pltpu.SemaphoreType.DMA,
        pltpu.SemaphoreType.DMA,
        pltpu.SemaphoreType.DMA,
        pltpu.SemaphoreType.DMA,
    ],
)
def _sc_agg(g_hbm, src_hbm, dst_hbm, out_hbm, sidx_all, didx_pg, rows0,
            rows1, acc_sh, sem0, sem1, psem0, psem1):
    cid = lax.axis_index("c")
    sid = lax.axis_index("s")
    wid = sid * NC + cid

    pltpu.async_copy(src_hbm.at[pl.ds(wid * EPW, EPW)], sidx_all, sem0)
    pltpu.async_copy(dst_hbm.at[wid, pl.ds(0, PG)], didx_pg.at[0], psem0)

    def zero_row(r, _):
        for j in range(D // 16):
            rows0[r, pl.ds(j * 16, 16)] = jnp.zeros((16,), jnp.float32)
        return 0

    lax.fori_loop(0, CHUNK, zero_row, 0)
    for i in range(ROWS_PT // CHUNK):
        pltpu.sync_copy(rows0, acc_sh.at[pl.ds(sid * ROWS_PT + i * CHUNK, CHUNK)])
    pltpu.make_async_copy(src_hbm.at[pl.ds(wid * EPW, EPW)], sidx_all,
                          sem0).wait()
    pltpu.make_async_copy(dst_hbm.at[wid, pl.ds(0, PG)], didx_pg.at[0],
                          psem0).wait()
    plsc.subcore_barrier()

    def sidx(k):
        return sidx_all.at[pl.ds(k * CHUNK, CHUNK)]

    pltpu.async_copy(g_hbm.at[sidx(0)], rows0, sem0)

    for p in range(NPAGE):
        pg = p % 2
        psem_nxt = psem1 if (1 - pg) else psem0
        if p + 1 < NPAGE:
            pltpu.async_copy(dst_hbm.at[wid, pl.ds((p + 1) * PG, PG)],
                             didx_pg.at[1 - pg], psem_nxt)
        if p > 0:
            psem_cur = psem1 if pg else psem0
            pltpu.make_async_copy(dst_hbm.at[wid, pl.ds(p * PG, PG)],
                                  didx_pg.at[pg], psem_cur).wait()

        npairs = PG // 2 if p + 1 < NPAGE else PG // 2 - 1

        def body(t, _, p=p, pg=pg):
            k0 = p * PG + 2 * t
            pltpu.async_copy(g_hbm.at[sidx(k0 + 1)], rows1, sem1)
            pltpu.make_async_copy(g_hbm.at[sidx(k0)], rows0, sem0).wait()
            pltpu.sync_copy(rows0, acc_sh.at[didx_pg.at[pg, 2 * t]], add=True)
            pltpu.async_copy(g_hbm.at[sidx(k0 + 2)], rows0, sem0)
            pltpu.make_async_copy(g_hbm.at[sidx(k0 + 1)], rows1, sem1).wait()
            pltpu.sync_copy(rows1, acc_sh.at[didx_pg.at[pg, 2 * t + 1]],
                            add=True)
            return 0

        lax.fori_loop(0, npairs, body, 0)

    lpg = (NPAGE - 1) % 2
    pltpu.async_copy(g_hbm.at[sidx(NCHUNK - 1)], rows1, sem1)
    pltpu.make_async_copy(g_hbm.at[sidx(NCHUNK - 2)], rows0, sem0).wait()
    pltpu.sync_copy(rows0, acc_sh.at[didx_pg.at[lpg, PG - 2]], add=True)
    pltpu.make_async_copy(g_hbm.at[sidx(NCHUNK - 1)], rows1, sem1).wait()
    pltpu.sync_copy(rows1, acc_sh.at[didx_pg.at[lpg, PG - 1]], add=True)

    plsc.subcore_barrier()
    pltpu.sync_copy(acc_sh.at[pl.ds(sid * ROWS_PT, ROWS_PT)],
                    out_hbm.at[cid, pl.ds(sid * ROWS_PT, ROWS_PT)])


BLK = 2000


def _dinv(degs_ref):
    return lax.rsqrt(degs_ref[...] + 1.0)


def _gemm_scale_body(x_ref, w_ref, degp_ref, g_ref):
    h = jnp.dot(x_ref[...], w_ref[...], preferred_element_type=jnp.float32)
    g_ref[...] = h * _dinv(degp_ref)


def _tc_gemm_scale(x, w, degp):
    return pl.pallas_call(
        _gemm_scale_body,
        grid=(N // BLK,),
        in_specs=[
            pl.BlockSpec((BLK, D), lambda i: (i, 0)),
            pl.BlockSpec((D, D), lambda i: (0, 0)),
            pl.BlockSpec((BLK, 1), lambda i: (i, 0)),
        ],
        out_specs=pl.BlockSpec((BLK, D), lambda i: (i, 0)),
        out_shape=jax.ShapeDtypeStruct((N, D), jnp.float32),
    )(x, w, degp)


def _mid_body(p_ref, g_ref, degp_ref, b_ref, w_ref, o_ref):
    dinv = _dinv(degp_ref)
    acc = p_ref[0] + p_ref[1] + g_ref[...]
    x2 = jnp.maximum(acc * dinv + b_ref[...], 0.0)
    h2 = jnp.dot(x2, w_ref[...], preferred_element_type=jnp.float32)
    o_ref[...] = h2 * dinv


def _tc_mid(parts, g, degp, b, w):
    return pl.pallas_call(
        _mid_body,
        grid=(N // BLK,),
        in_specs=[
            pl.BlockSpec((NC, BLK, D), lambda i: (0, i, 0)),
            pl.BlockSpec((BLK, D), lambda i: (i, 0)),
            pl.BlockSpec((BLK, 1), lambda i: (i, 0)),
            pl.BlockSpec((1, D), lambda i: (0, 0)),
            pl.BlockSpec((D, D), lambda i: (0, 0)),
        ],
        out_specs=pl.BlockSpec((BLK, D), lambda i: (i, 0)),
        out_shape=jax.ShapeDtypeStruct((N, D), jnp.float32),
    )(parts, g, degp, b, w)


def _final_body(p_ref, g_ref, degp_ref, b_ref, o_ref):
    acc = p_ref[0] + p_ref[1] + g_ref[...]
    o_ref[...] = acc * _dinv(degp_ref) + b_ref[...]


def _tc_final(parts, g, degp, b):
    return pl.pallas_call(
        _final_body,
        grid=(N // BLK,),
        in_specs=[
            pl.BlockSpec((NC, BLK, D), lambda i: (0, i, 0)),
            pl.BlockSpec((BLK, D), lambda i: (i, 0)),
            pl.BlockSpec((BLK, 1), lambda i: (i, 0)),
            pl.BlockSpec((1, D), lambda i: (0, 0)),
        ],
        out_specs=pl.BlockSpec((BLK, D), lambda i: (i, 0)),
        out_shape=jax.ShapeDtypeStruct((N, D), jnp.float32),
    )(parts, g, degp, b)


def kernel(edge_index, node_features, W1, b1, W2, b2):
    ppw = EPW - E // NW
    pad_src = jnp.broadcast_to(
        (jnp.arange(ppw, dtype=jnp.int32) * 41) % N, (NW, ppw))
    pad_dst = jnp.broadcast_to(
        N + (jnp.arange(ppw, dtype=jnp.int32) % (NP - N)), (NW, ppw))
    src = jnp.concatenate(
        [edge_index[0].reshape(NW, E // NW), pad_src], axis=1).reshape(-1)
    dst = jnp.concatenate(
        [edge_index[1].reshape(NW, E // NW), pad_dst], axis=1).reshape(
            NW, NCHUNK, CHUNK)

    degp = _sc_degree(dst)
    degs = (degp[0] + degp[1]).reshape(NP, 1)
    g1 = _tc_gemm_scale(node_features, W1, degs)
    p1 = _sc_agg(g1, src, dst)
    g2 = _tc_mid(p1, g1, degs, b1.reshape(1, D), W2)
    p2 = _sc_agg(g2, src, dst)
    return _tc_final(p2, g2, degs, b2.reshape(1, D))


# speedup vs baseline: 1.0327x; 1.0086x over previous
import functools

import jax
import jax.numpy as jnp
from jax import lax
from jax.experimental import pallas as pl
from jax.experimental.pallas import tpu as pltpu
from jax.experimental.pallas import tpu_sc as plsc

N = 10000
D = 128
E = 320000

NC = 2
NS = 16
NW = NC * NS
CHUNK = 128
NCHUNK = 80
EPW = NCHUNK * CHUNK
E_PAD = NW * EPW
NP = 10240
ROWS_PT = NP // NS
PG = 16
NPAGE = NCHUNK // PG

_sc_mesh = plsc.VectorSubcoreMesh(core_axis_name="c", subcore_axis_name="s")


@functools.partial(
    pl.kernel,
    out_type=jax.ShapeDtypeStruct((NC, NP), jnp.float32),
    mesh=_sc_mesh,
    scratch_types=[
        pltpu.VMEM((NCHUNK, CHUNK), jnp.int32),
        pltpu.VMEM((CHUNK,), jnp.float32),
        pltpu.VMEM((ROWS_PT,), jnp.float32),
        pltpu.VMEM_SHARED((NP,), jnp.float32),
        pltpu.SemaphoreType.DMA,
    ],
)
def _sc_degree(dst_hbm, out_hbm, idx_all, ones_v, zeros_v, deg_sh, sem):
    cid = lax.axis_index("c")
    sid = lax.axis_index("s")
    wid = sid * NC + cid

    pltpu.async_copy(dst_hbm.at[wid], idx_all, sem)
    for j in range(CHUNK // 16):
        ones_v[pl.ds(j * 16, 16)] = jnp.ones((16,), jnp.float32)
    for j in range(ROWS_PT // 16):
        zeros_v[pl.ds(j * 16, 16)] = jnp.zeros((16,), jnp.float32)
    pltpu.sync_copy(zeros_v, deg_sh.at[pl.ds(sid * ROWS_PT, ROWS_PT)])
    pltpu.make_async_copy(dst_hbm.at[wid], idx_all, sem).wait()
    plsc.subcore_barrier()

    def body(k, _):
        pltpu.sync_copy(ones_v, deg_sh.at[idx_all.at[k]], add=True)
        return 0

    lax.fori_loop(0, NCHUNK, body, 0)
    plsc.subcore_barrier()
    pltpu.sync_copy(deg_sh.at[pl.ds(sid * ROWS_PT, ROWS_PT)],
                    out_hbm.at[cid, pl.ds(sid * ROWS_PT, ROWS_PT)])


@functools.partial(
    pl.kernel,
    out_type=jax.ShapeDtypeStruct((NC, NP, D), jnp.float32),
    mesh=_sc_mesh,
    scratch_types=[
        pltpu.VMEM((EPW,), jnp.int32),
        pltpu.VMEM((2, PG, CHUNK), jnp.int32),
        pltpu.VMEM((CHUNK, D), jnp.float32),
        pltpu.VMEM((CHUNK, D), jnp.float32),
        pltpu.VMEM_SHARED((NP, D), jnp.float32),
        pltpu.SemaphoreType.DMA,
        pltpu.SemaphoreType.DMA,
        pltpu.SemaphoreType.DMA,
        pltpu.SemaphoreType.DMA,
    ],
)
def _sc_agg(g_hbm, src_hbm, dst_hbm, out_hbm, sidx_all, didx_pg, rows0,
            rows1, acc_sh, sem0, sem1, psem0, psem1):
    cid = lax.axis_index("c")
    sid = lax.axis_index("s")
    wid = sid * NC + cid

    pltpu.async_copy(src_hbm.at[pl.ds(wid * EPW, EPW)], sidx_all, sem0)
    pltpu.async_copy(dst_hbm.at[wid, pl.ds(0, PG)], didx_pg.at[0], psem0)

    def zero_row(r, _):
        for j in range(D // 16):
            rows0[r, pl.ds(j * 16, 16)] = jnp.zeros((16,), jnp.float32)
        return 0

    lax.fori_loop(0, CHUNK, zero_row, 0)
    for i in range(ROWS_PT // CHUNK):
        pltpu.sync_copy(rows0, acc_sh.at[pl.ds(sid * ROWS_PT + i * CHUNK, CHUNK)])
    pltpu.make_async_copy(src_hbm.at[pl.ds(wid * EPW, EPW)], sidx_all,
                          sem0).wait()
    pltpu.make_async_copy(dst_hbm.at[wid, pl.ds(0, PG)], didx_pg.at[0],
                          psem0).wait()
    plsc.subcore_barrier()

    def sidx(k):
        return sidx_all.at[pl.ds(k * CHUNK, CHUNK)]

    pltpu.async_copy(g_hbm.at[sidx(0)], rows0, sem0)

    for p in range(NPAGE):
        pg = p % 2
        psem_nxt = psem1 if (1 - pg) else psem0
        if p + 1 < NPAGE:
            pltpu.async_copy(dst_hbm.at[wid, pl.ds((p + 1) * PG, PG)],
                             didx_pg.at[1 - pg], psem_nxt)
        if p > 0:
            psem_cur = psem1 if pg else psem0
            pltpu.make_async_copy(dst_hbm.at[wid, pl.ds(p * PG, PG)],
                                  didx_pg.at[pg], psem_cur).wait()

        npairs = PG // 2 if p + 1 < NPAGE else PG // 2 - 1

        def body(t, _, p=p, pg=pg):
            k0 = p * PG + 2 * t
            pltpu.async_copy(g_hbm.at[sidx(k0 + 1)], rows1, sem1)
            pltpu.make_async_copy(g_hbm.at[sidx(k0)], rows0, sem0).wait()
            pltpu.sync_copy(rows0, acc_sh.at[didx_pg.at[pg, 2 * t]], add=True)
            pltpu.async_copy(g_hbm.at[sidx(k0 + 2)], rows0, sem0)
            pltpu.make_async_copy(g_hbm.at[sidx(k0 + 1)], rows1, sem1).wait()
            pltpu.sync_copy(rows1, acc_sh.at[didx_pg.at[pg, 2 * t + 1]],
                            add=True)
            return 0

        lax.fori_loop(0, npairs, body, 0)

    lpg = (NPAGE - 1) % 2
    pltpu.async_copy(g_hbm.at[sidx(NCHUNK - 1)], rows1, sem1)
    pltpu.make_async_copy(g_hbm.at[sidx(NCHUNK - 2)], rows0, sem0).wait()
    pltpu.sync_copy(rows0, acc_sh.at[didx_pg.at[lpg, PG - 2]], add=True)
    pltpu.make_async_copy(g_hbm.at[sidx(NCHUNK - 1)], rows1, sem1).wait()
    pltpu.sync_copy(rows1, acc_sh.at[didx_pg.at[lpg, PG - 1]], add=True)

    plsc.subcore_barrier()
    pltpu.sync_copy(acc_sh.at[pl.ds(sid * ROWS_PT, ROWS_PT)],
                    out_hbm.at[cid, pl.ds(sid * ROWS_PT, ROWS_PT)])


BLK = 5000


def _dinv(degs_ref):
    return lax.rsqrt(degs_ref[...] + 1.0)


def _gemm_scale_body(x_ref, w_ref, degp_ref, g_ref):
    h = jnp.dot(x_ref[...], w_ref[...], preferred_element_type=jnp.float32)
    g_ref[...] = h * _dinv(degp_ref)


def _tc_gemm_scale(x, w, degp):
    return pl.pallas_call(
        _gemm_scale_body,
        grid=(N // BLK,),
        in_specs=[
            pl.BlockSpec((BLK, D), lambda i: (i, 0)),
            pl.BlockSpec((D, D), lambda i: (0, 0)),
            pl.BlockSpec((BLK, 1), lambda i: (i, 0)),
        ],
        out_specs=pl.BlockSpec((BLK, D), lambda i: (i, 0)),
        out_shape=jax.ShapeDtypeStruct((N, D), jnp.float32),
    )(x, w, degp)


def _mid_body(p_ref, g_ref, degp_ref, b_ref, w_ref, o_ref):
    dinv = _dinv(degp_ref)
    acc = p_ref[0] + p_ref[1] + g_ref[...]
    x2 = jnp.maximum(acc * dinv + b_ref[...], 0.0)
    h2 = jnp.dot(x2, w_ref[...], preferred_element_type=jnp.float32)
    o_ref[...] = h2 * dinv


def _tc_mid(parts, g, degp, b, w):
    return pl.pallas_call(
        _mid_body,
        grid=(N // BLK,),
        in_specs=[
            pl.BlockSpec((NC, BLK, D), lambda i: (0, i, 0)),
            pl.BlockSpec((BLK, D), lambda i: (i, 0)),
            pl.BlockSpec((BLK, 1), lambda i: (i, 0)),
            pl.BlockSpec((1, D), lambda i: (0, 0)),
            pl.BlockSpec((D, D), lambda i: (0, 0)),
        ],
        out_specs=pl.BlockSpec((BLK, D), lambda i: (i, 0)),
        out_shape=jax.ShapeDtypeStruct((N, D), jnp.float32),
    )(parts, g, degp, b, w)


def _final_body(p_ref, g_ref, degp_ref, b_ref, o_ref):
    acc = p_ref[0] + p_ref[1] + g_ref[...]
    o_ref[...] = acc * _dinv(degp_ref) + b_ref[...]


def _tc_final(parts, g, degp, b):
    return pl.pallas_call(
        _final_body,
        grid=(N // BLK,),
        in_specs=[
            pl.BlockSpec((NC, BLK, D), lambda i: (0, i, 0)),
            pl.BlockSpec((BLK, D), lambda i: (i, 0)),
            pl.BlockSpec((BLK, 1), lambda i: (i, 0)),
            pl.BlockSpec((1, D), lambda i: (0, 0)),
        ],
        out_specs=pl.BlockSpec((BLK, D), lambda i: (i, 0)),
        out_shape=jax.ShapeDtypeStruct((N, D), jnp.float32),
    )(parts, g, degp, b)


def kernel(edge_index, node_features, W1, b1, W2, b2):
    ppw = EPW - E // NW
    pad_src = jnp.broadcast_to(
        (jnp.arange(ppw, dtype=jnp.int32) * 41) % N, (NW, ppw))
    pad_dst = jnp.broadcast_to(
        N + (jnp.arange(ppw, dtype=jnp.int32) % (NP - N)), (NW, ppw))
    src = jnp.concatenate(
        [edge_index[0].reshape(NW, E // NW), pad_src], axis=1).reshape(-1)
    dst = jnp.concatenate(
        [edge_index[1].reshape(NW, E // NW), pad_dst], axis=1).reshape(
            NW, NCHUNK, CHUNK)

    degp = _sc_degree(dst)
    degs = (degp[0] + degp[1]).reshape(NP, 1)
    g1 = _tc_gemm_scale(node_features, W1, degs)
    p1 = _sc_agg(g1, src, dst)
    g2 = _tc_mid(p1, g1, degs, b1.reshape(1, D), W2)
    p2 = _sc_agg(g2, src, dst)
    return _tc_final(p2, g2, degs, b2.reshape(1, D))
